# baseline (speedup 1.0000x reference)
import jax
import jax.numpy as jnp
from jax.experimental import pallas as pl

_F32 = jnp.float32


def _dg(a, b, dims):
    return jax.lax.dot_general(a, b, (dims, ((), ())),
                               preferred_element_type=_F32)


def _fused_kernel(x_ref, wgT_ref, asrc_ref, adst_ref, p_ref, bg_ref,
                  wp_ref, up_ref, bi_ref, bh_ref, m2_ref, b2_ref, out_ref):
    T, N, Fin = x_ref.shape
    xflat = x_ref[...].reshape(T * N, Fin)
    h_all = jnp.dot(xflat, wgT_ref[...], preferred_element_type=_F32)

    gts = []
    for t in range(T):
        ht = h_all[t * N:(t + 1) * N]
        a_s = _dg(ht, asrc_ref[...], ((1,), (1,)))
        a_d = _dg(adst_ref[...], ht, ((1,), (1,)))
        a_dp = _dg(a_d, p_ref[...], ((1,), (1,)))
        e = a_s + a_dp
        e = jnp.where(e > 0, e, 0.2 * e)
        m = jnp.max(e, axis=0, keepdims=True)
        ex = jnp.exp(e - m)
        s = jnp.sum(ex, axis=0, keepdims=True)
        gt = _dg(ht, ex, ((0,), (0,)))
        gts.append(gt / (s + 1e-16) + bg_ref[...])

    h = jnp.zeros((8, N), _F32)
    hs = []
    for tp in range(T):
        xt = jnp.concatenate([g[:, tp * 64:(tp + 1) * 64] for g in gts],
                             axis=1)
        gi = jnp.dot(wp_ref[...], xt, preferred_element_type=_F32) + bi_ref[...]
        gh = jnp.dot(up_ref[...], h, preferred_element_type=_F32) + bh_ref[...]
        r = jax.nn.sigmoid(gi[0:8] + gh[0:8])
        z = jax.nn.sigmoid(gi[8:16] + gh[8:16])
        n = jnp.tanh(gi[16:24] + r * gh[16:24])
        h = (1.0 - z) * n + z * h
        hs.append(h)

    zcol = jnp.zeros((8, 1), _F32)
    blocks = []
    for c in range(T):
        hc = hs[c]
        blocks.append(jnp.concatenate([zcol, hc[:, :-1]], axis=1))
        blocks.append(hc)
        blocks.append(jnp.concatenate([hc[:, 1:], zcol], axis=1))
    a_mat = jnp.concatenate(blocks, axis=0)
    out_ref[...] = _dg(m2_ref[...], a_mat, ((0,), (0,))) + b2_ref[...]


def kernel(x, adj_matrix, W_gat, att_src, att_dst, b_gat,
           W_ih, W_hh, b_ih, b_hh, W_conv, b_conv, W_out, b_out):
    B, T, N, Fin = x.shape
    H = W_gat.shape[0]
    O = W_conv.shape[0]

    r_idx = jnp.arange(N)
    rho = 8 * (r_idx % 64) + r_idx // 64
    P = (rho[:, None] == jnp.arange(N)[None, :]).astype(_F32)

    Wp = jnp.zeros((24, H), _F32)
    Wp = Wp.at[0:4].set(W_ih[0:4]).at[8:12].set(W_ih[4:8]).at[16:20].set(W_ih[8:12])
    Up = jnp.zeros((24, 8), _F32)
    Up = Up.at[0:4, 0:4].set(W_hh[0:4]).at[8:12, 0:4].set(W_hh[4:8]).at[16:20, 0:4].set(W_hh[8:12])
    bi = jnp.zeros((24, 1), _F32)
    bi = bi.at[0:4, 0].set(b_ih[0:4]).at[8:12, 0].set(b_ih[4:8]).at[16:20, 0].set(b_ih[8:12])
    bh = jnp.zeros((24, 1), _F32)
    bh = bh.at[0:4, 0].set(b_hh[0:4]).at[8:12, 0].set(b_hh[4:8]).at[16:20, 0].set(b_hh[8:12])

    Wt = jnp.transpose(W_conv, (1, 2, 3, 0))
    M = jnp.zeros((T, 3, 4, O, 4), _F32)
    for w in range(4):
        for dx in range(3):
            wi = w - 1 + dx
            if 0 <= wi < 4:
                M = M.at[:, :, wi, :, w].set(Wt[:, :, dx, :])
    M2 = jnp.einsum('cdiow,kw->cdiok', M, W_out).reshape(T, 3, 4, O * 2)
    M2p = jnp.zeros((T, 3, 8, O * 2), _F32).at[:, :, 0:4, :].set(M2)
    M2p = M2p.reshape(T * 3 * 8, O * 2)
    bias2 = (b_conv[:, None] * jnp.sum(W_out, axis=1)[None, :]
             + b_out[None, :]).reshape(O * 2, 1)

    F = pl.pallas_call(
        _fused_kernel,
        out_shape=jax.ShapeDtypeStruct((O * 2, N), _F32),
    )(x[0], W_gat.T, att_src[None, :], att_dst[None, :], P, b_gat[:, None],
      Wp, Up, bi, bh, M2p, bias2)

    return F.reshape(O, 2, N).transpose(0, 2, 1)[None]


# speedup vs baseline: 1.4319x; 1.4319x over previous
import jax
import jax.numpy as jnp
import numpy as np
from jax.experimental import pallas as pl

_F32 = jnp.float32

_N = 512
_rho = 8 * (np.arange(_N) % 64) + np.arange(_N) // 64
_P_NP = (_rho[:, None] == np.arange(_N)[None, :]).astype(np.float32)
_PAD24 = np.zeros((24, 12), np.float32)
for _g in range(3):
    for _j in range(4):
        _PAD24[_g * 8 + _j, _g * 4 + _j] = 1.0
_E48 = np.zeros((4, 8), np.float32)
_E48[:4, :4] = np.eye(4, dtype=np.float32)
_SEL = np.zeros((3, 4, 8), np.float32)
for _w in range(4):
    for _dx in range(3):
        _wi = _w - 1 + _dx
        if 0 <= _wi < 4:
            _SEL[_dx, _w, _wi] = 1.0


def _dg(a, b, dims):
    return jax.lax.dot_general(a, b, (dims, ((), ())),
                               preferred_element_type=_F32)


def _fused_kernel(x_ref, wgT_ref, asrc_ref, adst_ref, p_ref, bg_ref,
                  wp_ref, up_ref, bi_ref, bh_ref, m2_ref, b2_ref, out_ref):
    T, N, Fin = x_ref.shape
    xflat = x_ref[...].reshape(T * N, Fin)
    h_all = jnp.dot(xflat, wgT_ref[...], preferred_element_type=_F32)

    gts = []
    for t in range(T):
        ht = h_all[t * N:(t + 1) * N]
        a_s = _dg(ht, asrc_ref[...], ((1,), (1,)))
        a_d = _dg(adst_ref[...], ht, ((1,), (1,)))
        a_dp = _dg(a_d, p_ref[...], ((1,), (1,)))
        e = a_s + a_dp
        e = jnp.where(e > 0, e, 0.2 * e)
        m = jnp.max(e, axis=0, keepdims=True)
        ex = jnp.exp(e - m)
        s = jnp.sum(ex, axis=0, keepdims=True)
        gt = _dg(ht, ex, ((0,), (0,)))
        gts.append(gt / (s + 1e-16) + bg_ref[...])

    h = jnp.zeros((8, N), _F32)
    hs = []
    for tp in range(T):
        xt = jnp.concatenate([g[:, tp * 64:(tp + 1) * 64] for g in gts],
                             axis=1)
        gi = jnp.dot(wp_ref[...], xt, preferred_element_type=_F32) + bi_ref[...]
        gh = jnp.dot(up_ref[...], h, preferred_element_type=_F32) + bh_ref[...]
        r = jax.nn.sigmoid(gi[0:8] + gh[0:8])
        z = jax.nn.sigmoid(gi[8:16] + gh[8:16])
        n = jnp.tanh(gi[16:24] + r * gh[16:24])
        h = (1.0 - z) * n + z * h
        hs.append(h)

    zcol = jnp.zeros((8, 1), _F32)
    blocks = []
    for c in range(T):
        hc = hs[c]
        blocks.append(jnp.concatenate([zcol, hc[:, :-1]], axis=1))
        blocks.append(hc)
        blocks.append(jnp.concatenate([hc[:, 1:], zcol], axis=1))
    a_mat = jnp.concatenate(blocks, axis=0)
    out_ref[...] = _dg(m2_ref[...], a_mat, ((0,), (0,))) + b2_ref[...]


def kernel(x, adj_matrix, W_gat, att_src, att_dst, b_gat,
           W_ih, W_hh, b_ih, b_hh, W_conv, b_conv, W_out, b_out):
    B, T, N, Fin = x.shape
    H = W_gat.shape[0]
    O = W_conv.shape[0]

    Wp = _PAD24 @ W_ih
    Up = (_PAD24 @ W_hh) @ _E48
    bi = (_PAD24 @ b_ih)[:, None]
    bh = (_PAD24 @ b_hh)[:, None]

    M2p = jnp.einsum('ocdx,xwi,kw->cdiok', W_conv, _SEL, W_out)
    M2p = M2p.reshape(T * 3 * 8, O * 2)
    bias2 = (b_conv[:, None] * jnp.sum(W_out, axis=1)[None, :]
             + b_out[None, :]).reshape(O * 2, 1)

    F = pl.pallas_call(
        _fused_kernel,
        out_shape=jax.ShapeDtypeStruct((O * 2, N), _F32),
    )(x[0], W_gat.T, att_src[None, :], att_dst[None, :], jnp.asarray(_P_NP),
      b_gat[:, None], Wp, Up, bi, bh, M2p, bias2)

    return F.reshape(O, 2, N).transpose(0, 2, 1)[None]


# speedup vs baseline: 1.7012x; 1.1881x over previous
import jax
import jax.numpy as jnp
import numpy as np
from jax.experimental import pallas as pl

_F32 = jnp.float32

_N = 512
_rho = 8 * (np.arange(_N) % 64) + np.arange(_N) // 64
_P_NP = (_rho[:, None] == np.arange(_N)[None, :]).astype(np.float32)
_PAD24 = np.zeros((24, 12), np.float32)
for _g in range(3):
    for _j in range(4):
        _PAD24[_g * 8 + _j, _g * 4 + _j] = 1.0
_E48 = np.zeros((4, 8), np.float32)
_E48[:4, :4] = np.eye(4, dtype=np.float32)
_SEL = np.zeros((3, 4, 8), np.float32)
for _w in range(4):
    for _dx in range(3):
        _wi = _w - 1 + _dx
        if 0 <= _wi < 4:
            _SEL[_dx, _w, _wi] = 1.0


def _dg(a, b, dims):
    return jax.lax.dot_general(a, b, (dims, ((), ())),
                               preferred_element_type=_F32)


def _fused_kernel(x_ref, wg_ref, asrc_ref, adst_ref, p_ref, bg_ref,
                  wih_ref, whh_ref, bih_ref, bhh_ref, pad24_ref, e48_ref,
                  m2_ref, b2_ref, out_ref):
    _, T, N, Fin = x_ref.shape
    xflat = x_ref[0].reshape(T * N, Fin)
    h_all = _dg(xflat, wg_ref[...], ((1,), (1,)))

    pad24 = pad24_ref[...]
    wp = jnp.dot(pad24, wih_ref[...], preferred_element_type=_F32)
    up = jnp.dot(jnp.dot(pad24, whh_ref[...], preferred_element_type=_F32),
                 e48_ref[...], preferred_element_type=_F32)
    bi = (_dg(pad24, bih_ref[...], ((1,), (1,)))
          + _dg(wp, bg_ref[...], ((1,), (1,))))
    bh = _dg(pad24, bhh_ref[...], ((1,), (1,)))

    gts = []
    for t in range(T):
        ht = h_all[t * N:(t + 1) * N]
        a_s = _dg(ht, asrc_ref[...], ((1,), (1,)))
        a_d = _dg(adst_ref[...], ht, ((1,), (1,)))
        a_dp = _dg(a_d, p_ref[...], ((1,), (1,)))
        t1 = a_s + a_dp
        e = jnp.maximum(t1, 0.2 * t1)
        amax = jnp.max(a_s)
        t2 = amax + a_dp
        m = jnp.maximum(t2, 0.2 * t2)
        ex = jnp.exp(e - m)
        s = jnp.sum(ex, axis=0, keepdims=True)
        gt = _dg(ht, ex, ((0,), (0,)))
        gts.append(gt / (s + 1e-16))

    h = jnp.zeros((8, N), _F32)
    hs = []
    for tp in range(T):
        xt = jnp.concatenate([g[:, tp * 64:(tp + 1) * 64] for g in gts],
                             axis=1)
        gi = jnp.dot(wp, xt, preferred_element_type=_F32) + bi
        gh = jnp.dot(up, h, preferred_element_type=_F32) + bh
        r = jax.nn.sigmoid(gi[0:8] + gh[0:8])
        z = jax.nn.sigmoid(gi[8:16] + gh[8:16])
        n = jnp.tanh(gi[16:24] + r * gh[16:24])
        h = (1.0 - z) * n + z * h
        hs.append(h)

    zcol = jnp.zeros((8, 1), _F32)
    blocks = []
    for c in range(T):
        hc = hs[c]
        blocks.append(jnp.concatenate([zcol, hc[:, :-1]], axis=1))
        blocks.append(hc)
        blocks.append(jnp.concatenate([hc[:, 1:], zcol], axis=1))
    a_mat = jnp.concatenate(blocks, axis=0)
    out_ref[...] = _dg(m2_ref[...], a_mat, ((0,), (0,))) + b2_ref[...]


def kernel(x, adj_matrix, W_gat, att_src, att_dst, b_gat,
           W_ih, W_hh, b_ih, b_hh, W_conv, b_conv, W_out, b_out):
    B, T, N, Fin = x.shape
    O = W_conv.shape[0]

    M2p = jnp.einsum('ocdx,xwi,kw->cdiok', W_conv, _SEL, W_out)
    M2p = M2p.reshape(T * 3 * 8, O * 2)
    bias2 = (b_conv[:, None] * jnp.sum(W_out, axis=1)[None, :]
             + b_out[None, :]).reshape(O * 2, 1)

    F = pl.pallas_call(
        _fused_kernel,
        out_shape=jax.ShapeDtypeStruct((O * 2, N), _F32),
    )(x, W_gat, att_src[None, :], att_dst[None, :], jnp.asarray(_P_NP),
      b_gat[None, :], W_ih, W_hh, b_ih[None, :], b_hh[None, :],
      jnp.asarray(_PAD24), jnp.asarray(_E48), M2p, bias2)

    return F.reshape(O, 2, N).transpose(0, 2, 1)[None]


# speedup vs baseline: 2.1542x; 1.2663x over previous
import jax
import jax.numpy as jnp
import numpy as np
from jax.experimental import pallas as pl

_F32 = jnp.float32

_N = 512
_rho = 8 * (np.arange(_N) % 64) + np.arange(_N) // 64
_P_NP = (_rho[:, None] == np.arange(_N)[None, :]).astype(np.float32)
_PAD24 = np.zeros((24, 12), np.float32)
for _g in range(3):
    for _j in range(4):
        _PAD24[_g * 8 + _j, _g * 4 + _j] = 1.0
_E48 = np.zeros((4, 8), np.float32)
_E48[:4, :4] = np.eye(4, dtype=np.float32)
_SEL = np.zeros((3, 4, 8), np.float32)
for _w in range(4):
    for _dx in range(3):
        _wi = _w - 1 + _dx
        if 0 <= _wi < 4:
            _SEL[_dx, _w, _wi] = 1.0
_SEL2F = np.zeros((24, 4), np.float32)
for _x in range(3):
    for _i in range(8):
        _SEL2F[_x * 8 + _i] = _SEL[_x, :, _i]
_SELX = np.zeros((72, 72), np.float32)
for _x in range(3):
    for _c in range(8):
        for _d in range(3):
            _SELX[_x * 24 + _c * 3 + _d, _c * 9 + _d * 3 + _x] = 1.0
_RREP = np.kron(np.eye(24), np.ones((8, 1))).astype(np.float32)
_CREPT = np.kron(np.eye(12), np.ones((1, 2))).astype(np.float32)
_TILER = np.kron(np.ones((24, 1)), np.eye(8)).astype(np.float32)
_TILEC = np.kron(np.ones((1, 12)), np.eye(2)).astype(np.float32)
_R12 = np.kron(np.eye(12), np.ones((2, 1))).astype(np.float32)
_TILEB = np.kron(np.ones((12, 1)), np.eye(2)).astype(np.float32)
_ONES4 = np.ones((1, 4), np.float32)


def _dg(a, b, dims):
    return jax.lax.dot_general(a, b, (dims, ((), ())),
                               preferred_element_type=_F32)


def _fused_kernel(x_ref, wg_ref, asrc_ref, adst_ref, p_ref, bg_ref,
                  wih_ref, whh_ref, bih_ref, bhh_ref, pad24_ref, e48_ref,
                  wcf_ref, wout_ref, bconv_ref, bout_ref,
                  sel2f_ref, selx_ref, rrep_ref, crept_ref, tiler_ref,
                  tilec_ref, r12_ref, tileb_ref, ones4_ref, out_ref):
    _, T, N, Fin = x_ref.shape
    xflat = x_ref[0].reshape(T * N, Fin)
    h_all = _dg(xflat, wg_ref[...], ((1,), (1,)))

    pad24 = pad24_ref[...]
    wp = jnp.dot(pad24, wih_ref[...], preferred_element_type=_F32)
    up = jnp.dot(jnp.dot(pad24, whh_ref[...], preferred_element_type=_F32),
                 e48_ref[...], preferred_element_type=_F32)
    bi = (_dg(pad24, bih_ref[...], ((1,), (1,)))
          + _dg(wp, bg_ref[...], ((1,), (1,))))
    bh = _dg(pad24, bhh_ref[...], ((1,), (1,)))

    t2f = _dg(sel2f_ref[...], wout_ref[...], ((1,), (1,)))
    m2 = None
    for xtap in range(3):
        a_x = _dg(selx_ref[24 * xtap:24 * (xtap + 1)], wcf_ref[...],
                  ((1,), (1,)))
        a_exp = jnp.dot(rrep_ref[...],
                        jnp.dot(a_x, crept_ref[...],
                                preferred_element_type=_F32),
                        preferred_element_type=_F32)
        b_exp = jnp.dot(jnp.dot(tiler_ref[...], t2f[8 * xtap:8 * xtap + 8],
                                preferred_element_type=_F32),
                        tilec_ref[...], preferred_element_type=_F32)
        term = a_exp * b_exp
        m2 = term if m2 is None else m2 + term
    sw = _dg(wout_ref[...], ones4_ref[...], ((1,), (1,)))
    b2 = (_dg(r12_ref[...], bconv_ref[...], ((1,), (1,)))
          * jnp.dot(tileb_ref[...], sw, preferred_element_type=_F32)
          + _dg(tileb_ref[...], bout_ref[...], ((1,), (1,))))

    hts, rows = [], []
    for t in range(T):
        ht = h_all[t * N:(t + 1) * N]
        a_s = _dg(ht, asrc_ref[...], ((1,), (1,)))
        a_d = _dg(adst_ref[...], ht, ((1,), (1,)))
        a_dp = _dg(a_d, p_ref[...], ((1,), (1,)))
        amax = jnp.max(a_s)
        t2 = amax + a_dp
        m = jnp.maximum(t2, 0.2 * t2)
        hts.append(ht)
        rows.append((a_s, a_dp, m))
    exs = []
    for t in range(T):
        a_s, a_dp, m = rows[t]
        t1 = a_s + a_dp
        e = jnp.maximum(t1, 0.2 * t1)
        ex = jnp.exp(e - m)
        exs.append(ex)
    gts = []
    for t in range(T):
        ex = exs[t]
        s = jnp.sum(ex, axis=0, keepdims=True)
        gt = _dg(hts[t], ex, ((0,), (0,)))
        gts.append(gt / (s + 1e-16))

    h = jnp.zeros((8, N), _F32)
    hs = []
    for tp in range(T):
        xt = jnp.concatenate([g[:, tp * 64:(tp + 1) * 64] for g in gts],
                             axis=1)
        gi = jnp.dot(wp, xt, preferred_element_type=_F32) + bi
        gh = jnp.dot(up, h, preferred_element_type=_F32) + bh
        r = jax.nn.sigmoid(gi[0:8] + gh[0:8])
        z = jax.nn.sigmoid(gi[8:16] + gh[8:16])
        n = jnp.tanh(gi[16:24] + r * gh[16:24])
        h = (1.0 - z) * n + z * h
        hs.append(h)

    zcol = jnp.zeros((8, 1), _F32)
    blocks = []
    for c in range(T):
        hc = hs[c]
        blocks.append(jnp.concatenate([zcol, hc[:, :-1]], axis=1))
        blocks.append(hc)
        blocks.append(jnp.concatenate([hc[:, 1:], zcol], axis=1))
    a_mat = jnp.concatenate(blocks, axis=0)
    out_ref[...] = _dg(m2, a_mat, ((0,), (0,))) + b2


def kernel(x, adj_matrix, W_gat, att_src, att_dst, b_gat,
           W_ih, W_hh, b_ih, b_hh, W_conv, b_conv, W_out, b_out):
    B, T, N, Fin = x.shape
    O = W_conv.shape[0]

    F = pl.pallas_call(
        _fused_kernel,
        out_shape=jax.ShapeDtypeStruct((O * 2, N), _F32),
    )(x, W_gat, att_src[None, :], att_dst[None, :], jnp.asarray(_P_NP),
      b_gat[None, :], W_ih, W_hh, b_ih[None, :], b_hh[None, :],
      jnp.asarray(_PAD24), jnp.asarray(_E48),
      W_conv.reshape(O, 72), W_out, b_conv[None, :], b_out[None, :],
      jnp.asarray(_SEL2F), jnp.asarray(_SELX), jnp.asarray(_RREP),
      jnp.asarray(_CREPT), jnp.asarray(_TILER), jnp.asarray(_TILEC),
      jnp.asarray(_R12), jnp.asarray(_TILEB), jnp.asarray(_ONES4))

    return F.reshape(O, 2, N).transpose(0, 2, 1)[None]
